# initial kernel scaffold (unmeasured)
import functools

import jax
import jax.numpy as jnp
from jax import lax
from jax.experimental import pallas as pl
from jax.experimental.pallas import tpu as pltpu

B, S, H, Dh, Dr = 4, 256, 32, 128, 64
D = 4096
DC_HALF = 128
M = B * S
F32 = jnp.float32


def _exchange(x2d, Wdkv, Wuk, Wuv, Wkr):

    def body(x_ref, wdkv_ref, wuk_ref, wuv_ref, wkr_ref,
             c_ref, wukc_ref, wuvc_ref, kr_ref,
             send_sems, recv_sems):
        my_x = lax.axis_index("x")
        my_y = lax.axis_index("y")
        my_z = lax.axis_index("z")
        peer = (my_x, 1 - my_y, my_z)

        barrier_sem = pltpu.get_barrier_semaphore()
        pl.semaphore_signal(barrier_sem, inc=1, device_id=peer,
                            device_id_type=pl.DeviceIdType.MESH)
        pl.semaphore_wait(barrier_sem, 1)

        wuk_rdma = pltpu.make_async_remote_copy(
            src_ref=wuk_ref, dst_ref=wukc_ref.at[1],
            send_sem=send_sems.at[0], recv_sem=recv_sems.at[0],
            device_id=peer, device_id_type=pl.DeviceIdType.MESH)
        wuk_rdma.start()
        wuv_rdma = pltpu.make_async_remote_copy(
            src_ref=wuv_ref, dst_ref=wuvc_ref.at[1],
            send_sem=send_sems.at[1], recv_sem=recv_sems.at[1],
            device_id=peer, device_id_type=pl.DeviceIdType.MESH)
        wuv_rdma.start()

        c_ref[0] = jnp.dot(x_ref[...], wdkv_ref[...],
                           preferred_element_type=F32)
        kr_ref[...] = jnp.dot(x_ref[...], wkr_ref[...],
                              preferred_element_type=F32)
        wukc_ref[0] = wuk_ref[...]
        wuvc_ref[0] = wuv_ref[...]

        c_rdma = pltpu.make_async_remote_copy(
            src_ref=c_ref.at[0], dst_ref=c_ref.at[1],
            send_sem=send_sems.at[2], recv_sem=recv_sems.at[2],
            device_id=peer, device_id_type=pl.DeviceIdType.MESH)
        c_rdma.start()

        wuk_rdma.wait()
        wuv_rdma.wait()
        c_rdma.wait()

    return pl.pallas_call(
        body,
        out_shape=[
            jax.ShapeDtypeStruct((2, M, DC_HALF), F32),
            jax.ShapeDtypeStruct((2, DC_HALF, D), F32),
            jax.ShapeDtypeStruct((2, DC_HALF, D), F32),
            jax.ShapeDtypeStruct((M, Dr), F32),
        ],
        in_specs=[pl.BlockSpec(memory_space=pltpu.VMEM)] * 5,
        out_specs=[pl.BlockSpec(memory_space=pltpu.VMEM)] * 4,
        scratch_shapes=[
            pltpu.SemaphoreType.DMA((3,)),
            pltpu.SemaphoreType.DMA((3,)),
        ],
        compiler_params=pltpu.CompilerParams(collective_id=0),
    )(x2d, Wdkv, Wuk, Wuv, Wkr)


def _kv(c_cat, wuk_cat, wuv_cat):

    def body(c_ref, wuk_ref, wuv_ref, k_ref, v_ref):
        c0 = c_ref[0]
        c1 = c_ref[1]
        k_ref[...] = (jnp.dot(c0, wuk_ref[0], preferred_element_type=F32)
                      + jnp.dot(c1, wuk_ref[1], preferred_element_type=F32))
        v_ref[...] = (jnp.dot(c0, wuv_ref[0], preferred_element_type=F32)
                      + jnp.dot(c1, wuv_ref[1], preferred_element_type=F32))

    return pl.pallas_call(
        body,
        out_shape=[
            jax.ShapeDtypeStruct((M, D), F32),
            jax.ShapeDtypeStruct((M, D), F32),
        ],
        in_specs=[pl.BlockSpec(memory_space=pltpu.VMEM)] * 3,
        out_specs=[pl.BlockSpec(memory_space=pltpu.VMEM)] * 2,
    )(c_cat, wuk_cat, wuv_cat)


def _matmul_nt(a, b, n_tile):
    m, k = a.shape
    _, n = b.shape

    def body(a_ref, b_ref, o_ref):
        o_ref[...] = jnp.dot(a_ref[...], b_ref[...],
                             preferred_element_type=F32)

    return pl.pallas_call(
        body,
        grid=(n // n_tile,),
        in_specs=[
            pl.BlockSpec((m, k), lambda j: (0, 0)),
            pl.BlockSpec((k, n_tile), lambda j: (0, j)),
        ],
        out_specs=pl.BlockSpec((m, n_tile), lambda j: (0, j)),
        out_shape=jax.ShapeDtypeStruct((m, n), F32),
    )(a, b)


def _attn(q4, k4, v4, qr4, kr3):
    scale = (Dh + Dr) ** -0.5
    contract_last = (((1,), (1,)), ((), ()))

    def body(q_ref, k_ref, v_ref, qr_ref, kr_ref, o_ref):
        q = q_ref[0, :, 0, :]
        k = k_ref[0, :, 0, :]
        v = v_ref[0, :, 0, :]
        qr = qr_ref[0, :, 0, :]
        kr = kr_ref[0, :, :]
        s = (lax.dot_general(q, k, contract_last, preferred_element_type=F32)
             + lax.dot_general(qr, kr, contract_last,
                               preferred_element_type=F32)) * scale
        m = jnp.max(s, axis=-1, keepdims=True)
        p = jnp.exp(s - m)
        p = p / jnp.sum(p, axis=-1, keepdims=True)
        o_ref[0, :, 0, :] = jnp.dot(p, v, preferred_element_type=F32)

    hd_spec = pl.BlockSpec((1, S, 1, Dh), lambda b, h: (b, 0, h, 0))
    return pl.pallas_call(
        body,
        grid=(B, H),
        in_specs=[
            hd_spec,
            hd_spec,
            hd_spec,
            pl.BlockSpec((1, S, 1, Dr), lambda b, h: (b, 0, h, 0)),
            pl.BlockSpec((1, S, Dr), lambda b, h: (b, 0, 0)),
        ],
        out_specs=hd_spec,
        out_shape=jax.ShapeDtypeStruct((B, S, H, Dh), F32),
    )(q4, k4, v4, qr4, kr3)


def kernel(x, Wdkv, Wuk, Wuv, Wq, Wqr, Wkr, Wo):
    x2d = x.reshape(M, D)
    c_cat, wuk_cat, wuv_cat, kr = _exchange(x2d, Wdkv, Wuk, Wuv, Wkr)
    k, v = _kv(c_cat, wuk_cat, wuv_cat)
    q = _matmul_nt(x2d, Wq, 512)
    qr = _matmul_nt(x2d, Wqr, 512)
    o = _attn(
        q.reshape(B, S, H, Dh),
        k.reshape(B, S, H, Dh),
        v.reshape(B, S, H, Dh),
        qr.reshape(B, S, H, Dr),
        kr.reshape(B, S, Dr),
    )
    out = _matmul_nt(o.reshape(M, H * Dh), Wo, 512)
    return out.reshape(B, S, D)


# baseline (device time: 283529 ns/iter reference)
import functools

import jax
import jax.numpy as jnp
from jax import lax
from jax.experimental import pallas as pl
from jax.experimental.pallas import tpu as pltpu

B, S, H, Dh, Dr = 4, 256, 32, 128, 64
D = 4096
DC_HALF = 128
M = B * S
F32 = jnp.float32
_VMEM_LIMIT = 60 * 1024 * 1024


def _exchange(x2d, Wdkv, Wuk, Wuv, Wkr):

    def body(x_ref, wdkv_ref, wuk_ref, wuv_ref, wkr_ref,
             c_ref, wukc_ref, wuvc_ref, kr_ref,
             send_sems, recv_sems):
        my_x = lax.axis_index("x")
        my_y = lax.axis_index("y")
        my_z = lax.axis_index("z")
        peer = (my_x, 1 - my_y, my_z)

        barrier_sem = pltpu.get_barrier_semaphore()
        pl.semaphore_signal(barrier_sem, inc=1, device_id=peer,
                            device_id_type=pl.DeviceIdType.MESH)
        pl.semaphore_wait(barrier_sem, 1)

        wuk_rdma = pltpu.make_async_remote_copy(
            src_ref=wuk_ref, dst_ref=wukc_ref.at[1],
            send_sem=send_sems.at[0], recv_sem=recv_sems.at[0],
            device_id=peer, device_id_type=pl.DeviceIdType.MESH)
        wuk_rdma.start()
        wuv_rdma = pltpu.make_async_remote_copy(
            src_ref=wuv_ref, dst_ref=wuvc_ref.at[1],
            send_sem=send_sems.at[1], recv_sem=recv_sems.at[1],
            device_id=peer, device_id_type=pl.DeviceIdType.MESH)
        wuv_rdma.start()

        c_ref[0] = jnp.dot(x_ref[...], wdkv_ref[...],
                           preferred_element_type=F32)
        kr_ref[...] = jnp.dot(x_ref[...], wkr_ref[...],
                              preferred_element_type=F32)
        wukc_ref[0] = wuk_ref[...]
        wuvc_ref[0] = wuv_ref[...]

        c_rdma = pltpu.make_async_remote_copy(
            src_ref=c_ref.at[0], dst_ref=c_ref.at[1],
            send_sem=send_sems.at[2], recv_sem=recv_sems.at[2],
            device_id=peer, device_id_type=pl.DeviceIdType.MESH)
        c_rdma.start()

        wuk_rdma.wait()
        wuv_rdma.wait()
        c_rdma.wait()

    return pl.pallas_call(
        body,
        out_shape=[
            jax.ShapeDtypeStruct((2, M, DC_HALF), F32),
            jax.ShapeDtypeStruct((2, DC_HALF, D), F32),
            jax.ShapeDtypeStruct((2, DC_HALF, D), F32),
            jax.ShapeDtypeStruct((M, Dr), F32),
        ],
        in_specs=[pl.BlockSpec(memory_space=pltpu.VMEM)] * 5,
        out_specs=[pl.BlockSpec(memory_space=pltpu.VMEM)] * 4,
        scratch_shapes=[
            pltpu.SemaphoreType.DMA((3,)),
            pltpu.SemaphoreType.DMA((3,)),
        ],
        compiler_params=pltpu.CompilerParams(
            collective_id=0, vmem_limit_bytes=_VMEM_LIMIT),
    )(x2d, Wdkv, Wuk, Wuv, Wkr)


def _kv(c_cat, wuk_cat, wuv_cat):

    def body(c_ref, wuk_ref, wuv_ref, k_ref, v_ref):
        c0 = c_ref[0]
        c1 = c_ref[1]
        k_ref[...] = (jnp.dot(c0, wuk_ref[0], preferred_element_type=F32)
                      + jnp.dot(c1, wuk_ref[1], preferred_element_type=F32))
        v_ref[...] = (jnp.dot(c0, wuv_ref[0], preferred_element_type=F32)
                      + jnp.dot(c1, wuv_ref[1], preferred_element_type=F32))

    return pl.pallas_call(
        body,
        out_shape=[
            jax.ShapeDtypeStruct((M, D), F32),
            jax.ShapeDtypeStruct((M, D), F32),
        ],
        in_specs=[pl.BlockSpec(memory_space=pltpu.VMEM)] * 3,
        out_specs=[pl.BlockSpec(memory_space=pltpu.VMEM)] * 2,
        compiler_params=pltpu.CompilerParams(vmem_limit_bytes=_VMEM_LIMIT),
    )(c_cat, wuk_cat, wuv_cat)


def _matmul_nt(a, b, n_tile):
    m, k = a.shape
    _, n = b.shape

    def body(a_ref, b_ref, o_ref):
        o_ref[...] = jnp.dot(a_ref[...], b_ref[...],
                             preferred_element_type=F32)

    return pl.pallas_call(
        body,
        grid=(n // n_tile,),
        in_specs=[
            pl.BlockSpec((m, k), lambda j: (0, 0)),
            pl.BlockSpec((k, n_tile), lambda j: (0, j)),
        ],
        out_specs=pl.BlockSpec((m, n_tile), lambda j: (0, j)),
        out_shape=jax.ShapeDtypeStruct((m, n), F32),
        compiler_params=pltpu.CompilerParams(vmem_limit_bytes=_VMEM_LIMIT),
    )(a, b)


def _attn(q3, k3, v3, qr3, kr3):
    scale = (Dh + Dr) ** -0.5
    contract_last = (((1,), (1,)), ((), ()))

    def body(q_ref, k_ref, v_ref, qr_ref, kr_ref, o_ref):
        kr = kr_ref[0, :, :]
        for h in range(H):
            q = q_ref[0, :, h * Dh:(h + 1) * Dh]
            k = k_ref[0, :, h * Dh:(h + 1) * Dh]
            v = v_ref[0, :, h * Dh:(h + 1) * Dh]
            qr = qr_ref[0, :, h * Dr:(h + 1) * Dr]
            s = (lax.dot_general(q, k, contract_last,
                                 preferred_element_type=F32)
                 + lax.dot_general(qr, kr, contract_last,
                                   preferred_element_type=F32)) * scale
            m = jnp.max(s, axis=-1, keepdims=True)
            p = jnp.exp(s - m)
            p = p / jnp.sum(p, axis=-1, keepdims=True)
            o_ref[0, :, h * Dh:(h + 1) * Dh] = jnp.dot(
                p, v, preferred_element_type=F32)

    return pl.pallas_call(
        body,
        grid=(B,),
        in_specs=[
            pl.BlockSpec((1, S, H * Dh), lambda b: (b, 0, 0)),
            pl.BlockSpec((1, S, H * Dh), lambda b: (b, 0, 0)),
            pl.BlockSpec((1, S, H * Dh), lambda b: (b, 0, 0)),
            pl.BlockSpec((1, S, H * Dr), lambda b: (b, 0, 0)),
            pl.BlockSpec((1, S, Dr), lambda b: (b, 0, 0)),
        ],
        out_specs=pl.BlockSpec((1, S, H * Dh), lambda b: (b, 0, 0)),
        out_shape=jax.ShapeDtypeStruct((B, S, H * Dh), F32),
        compiler_params=pltpu.CompilerParams(vmem_limit_bytes=_VMEM_LIMIT),
    )(q3, k3, v3, qr3, kr3)


def kernel(x, Wdkv, Wuk, Wuv, Wq, Wqr, Wkr, Wo):
    x2d = x.reshape(M, D)
    c_cat, wuk_cat, wuv_cat, kr = _exchange(x2d, Wdkv, Wuk, Wuv, Wkr)
    k, v = _kv(c_cat, wuk_cat, wuv_cat)
    q = _matmul_nt(x2d, Wq, 512)
    qr = _matmul_nt(x2d, Wqr, 512)
    o = _attn(
        q.reshape(B, S, H * Dh),
        k.reshape(B, S, H * Dh),
        v.reshape(B, S, H * Dh),
        qr.reshape(B, S, H * Dr),
        kr.reshape(B, S, Dr),
    )
    out = _matmul_nt(o.reshape(M, H * Dh), Wo, 512)
    return out.reshape(B, S, D)
